# baseline (speedup 1.0000x reference)
import functools

import jax
import jax.numpy as jnp
from jax import lax
from jax.experimental import pallas as pl
from jax.experimental.pallas import tpu as pltpu
from jax.experimental.pallas import tpu_sc as plsc

K = 8
L = 16
NC = 2
NS = 16
CH = 16
WARM = 1

_SORT8 = (
    (0, 1), (2, 3), (4, 5), (6, 7),
    (0, 2), (1, 3), (4, 6), (5, 7),
    (1, 2), (5, 6),
    (0, 4), (1, 5), (2, 6), (3, 7),
    (2, 4), (3, 5),
    (1, 2), (3, 4), (5, 6),
)


def _sort8(vs):
    vs = list(vs)
    for i, j in _SORT8:
        hi = jnp.maximum(vs[i], vs[j])
        lo = jnp.minimum(vs[i], vs[j])
        vs[i], vs[j] = hi, lo
    return vs


def _merge_top8(r, v):
    m = [jnp.maximum(r[i], v[7 - i]) for i in range(K)]
    for i, j in ((0, 4), (1, 5), (2, 6), (3, 7),
                 (0, 2), (1, 3), (4, 6), (5, 7),
                 (0, 1), (2, 3), (4, 5), (6, 7)):
        hi = jnp.maximum(m[i], m[j])
        lo = jnp.minimum(m[i], m[j])
        m[i], m[j] = hi, lo
    return tuple(m)


def kernel(inputs):
    x = inputs
    B, S, D = x.shape
    NW = NC * NS
    CB = NW // B
    CW = D // CB
    NG = CW // L
    T = 256
    NT = S // T
    NB = T // K
    NCH = T // CH

    mesh = plsc.VectorSubcoreMesh(core_axis_name="c", subcore_axis_name="s")

    @functools.partial(
        pl.kernel,
        out_type=jax.ShapeDtypeStruct((NW, CW * K), jnp.float32),
        mesh=mesh,
        compiler_params=pltpu.CompilerParams(needs_layout_passes=False),
        scratch_types=[
            pltpu.VMEM((T + CH, CW), jnp.float32),
            pltpu.VMEM((T + CH, CW), jnp.float32),
            pltpu.VMEM((NCH * L,), jnp.int32),
            pltpu.VMEM((CW * K,), jnp.float32),
            pltpu.SemaphoreType.DMA,
            pltpu.SemaphoreType.DMA,
        ],
    )
    def kmax(x_hbm, out_hbm, buf0, buf1, clist, outv, sem0, sem1):
        wid = lax.axis_index("s") * NC + lax.axis_index("c")
        b = wid // CB
        c0 = (wid % CB) * CW

        iot = lax.broadcasted_iota(jnp.int32, (L,), 0)
        neg = jnp.full((L,), -jnp.inf, jnp.float32)

        for buf in (buf0, buf1):
            for r in range(CH):
                for g in range(NG):
                    buf[T + r, pl.ds(g * L, L)] = neg

        def src(t):
            return x_hbm.at[b, pl.ds(t * T, T), pl.ds(c0, CW)]

        def tile_group_dense(buf, g, rs):
            def blk_body(blk, rs, _g=g):
                s0 = blk * K
                vs = _sort8(buf[s0 + j, pl.ds(_g * L, L)] for j in range(K))
                return _merge_top8(rs, vs)
            return lax.fori_loop(0, NB, blk_body, rs)

        def tile_group_filtered(buf, g, rs):
            col = g * L
            r7 = rs[7]

            for c in range(NCH):
                clist[pl.ds(c * L, L)] = jnp.full((L,), NCH, jnp.int32)

            def chunk_scan(c, w):
                m = buf[c * CH, pl.ds(col, L)]
                for r in range(1, CH):
                    m = jnp.maximum(m, buf[c * CH + r, pl.ds(col, L)])
                qual = m > r7
                cvec = jnp.full((L,), 1, jnp.int32) * c
                plsc.store_scatter(clist, [w], cvec, mask=qual)
                return w + jnp.where(qual, L, 0)

            w = lax.fori_loop(0, NCH, chunk_scan, iot)
            jmax = lax.reduce_max(w, axes=(0,)) // L

            col_idx = iot + col

            def visit(j, rs):
                cid = clist[pl.ds(j * L, L)]
                row0 = cid * CH
                va = _sort8(
                    plsc.load_gather(buf, [row0 + r, col_idx])
                    for r in range(K))
                vb = _sort8(
                    plsc.load_gather(buf, [row0 + K + r, col_idx])
                    for r in range(K))
                return _merge_top8(rs, _merge_top8(va, vb))

            return lax.fori_loop(0, jmax, visit, rs)

        def process(buf, state, tile_group):
            return tuple(
                tile_group(buf, g, state[g]) for g in range(NG))

        def pair_body(tt, state, tile_group):
            t0 = tt * 2
            pltpu.async_copy(src(t0 + 1), buf1.at[pl.ds(0, T)], sem1)
            pltpu.make_async_copy(src(t0), buf0.at[pl.ds(0, T)], sem0).wait()
            state = process(buf0, state, tile_group)

            @pl.when(tt + 1 < NT // 2)
            def _():
                pltpu.async_copy(src(t0 + 2), buf0.at[pl.ds(0, T)], sem0)

            pltpu.make_async_copy(src(t0 + 1), buf1.at[pl.ds(0, T)], sem1).wait()
            return process(buf1, state, tile_group)

        pltpu.async_copy(src(0), buf0.at[pl.ds(0, T)], sem0)
        state = tuple(tuple(neg for _ in range(K)) for _ in range(NG))
        for tt in range(WARM):
            state = pair_body(tt, state, tile_group_dense)
        state = lax.fori_loop(
            WARM, NT // 2,
            lambda tt, st: pair_body(tt, st, tile_group_filtered),
            state)

        for g in range(NG):
            for i in range(K):
                outv[pl.ds(i * CW + g * L, L)] = state[g][i]
        pltpu.sync_copy(outv, out_hbm.at[wid])

    out = kmax(x)
    out = out.reshape(B, CB, K, CW).transpose(0, 1, 3, 2)
    return out.reshape(B, D * K)


# speedup vs baseline: 1.2201x; 1.2201x over previous
import functools

import jax
import jax.numpy as jnp
from jax import lax
from jax.experimental import pallas as pl
from jax.experimental.pallas import tpu as pltpu
from jax.experimental.pallas import tpu_sc as plsc

K = 8
L = 16
NC = 2
NS = 16

_SORT8 = (
    (0, 1), (2, 3), (4, 5), (6, 7),
    (0, 2), (1, 3), (4, 6), (5, 7),
    (1, 2), (5, 6),
    (0, 4), (1, 5), (2, 6), (3, 7),
    (2, 4), (3, 5),
    (1, 2), (3, 4), (5, 6),
)


def _sort8(vs):
    vs = list(vs)
    for i, j in _SORT8:
        hi = jnp.maximum(vs[i], vs[j])
        lo = jnp.minimum(vs[i], vs[j])
        vs[i], vs[j] = hi, lo
    return vs


def _merge_top8(r, v):
    m = [jnp.maximum(r[i], v[7 - i]) for i in range(K)]
    for i, j in ((0, 4), (1, 5), (2, 6), (3, 7),
                 (0, 2), (1, 3), (4, 6), (5, 7),
                 (0, 1), (2, 3), (4, 5), (6, 7)):
        hi = jnp.maximum(m[i], m[j])
        lo = jnp.minimum(m[i], m[j])
        m[i], m[j] = hi, lo
    return tuple(m)


def kernel(inputs):
    x = inputs
    B, S, D = x.shape
    NW = NC * NS
    CB = NW // B
    CW = D // CB
    NG = CW // L
    T = 256
    NT = S // T
    NB = T // K

    mesh = plsc.VectorSubcoreMesh(core_axis_name="c", subcore_axis_name="s")

    @functools.partial(
        pl.kernel,
        out_type=jax.ShapeDtypeStruct((NW, CW * K), jnp.float32),
        mesh=mesh,
        compiler_params=pltpu.CompilerParams(needs_layout_passes=False),
        scratch_types=[
            pltpu.VMEM((T, CW), jnp.float32),
            pltpu.VMEM((T, CW), jnp.float32),
            pltpu.VMEM((CW * K,), jnp.float32),
            pltpu.SemaphoreType.DMA,
            pltpu.SemaphoreType.DMA,
        ],
    )
    def kmax(x_hbm, out_hbm, buf0, buf1, outv, sem0, sem1):
        wid = lax.axis_index("s") * NC + lax.axis_index("c")
        b = wid // CB
        c0 = (wid % CB) * CW

        neg = jnp.full((L,), -jnp.inf, jnp.float32)

        def src(t):
            return x_hbm.at[b, pl.ds(t * T, T), pl.ds(c0, CW)]

        def process(buf, state):
            new_state = list(state)
            for g0 in range(0, NG, 2):
                def blk_body(blk, rs2, _g0=g0):
                    s0 = blk * K
                    ra, rb = rs2
                    va = _sort8(buf[s0 + j, pl.ds(_g0 * L, L)] for j in range(K))
                    vb = _sort8(
                        buf[s0 + j, pl.ds((_g0 + 1) * L, L)] for j in range(K))
                    return (_merge_top8(ra, va), _merge_top8(rb, vb))
                ra, rb = lax.fori_loop(
                    0, NB, blk_body, (state[g0], state[g0 + 1]))
                new_state[g0] = ra
                new_state[g0 + 1] = rb
            return tuple(new_state)

        def pair_body(tt, state):
            t0 = tt * 2
            pltpu.async_copy(src(t0 + 1), buf1, sem1)
            pltpu.make_async_copy(src(t0), buf0, sem0).wait()
            state = process(buf0, state)

            @pl.when(tt + 1 < NT // 2)
            def _():
                pltpu.async_copy(src(t0 + 2), buf0, sem0)

            pltpu.make_async_copy(src(t0 + 1), buf1, sem1).wait()
            return process(buf1, state)

        pltpu.async_copy(src(0), buf0, sem0)
        init = tuple(tuple(neg for _ in range(K)) for _ in range(NG))
        state = lax.fori_loop(0, NT // 2, pair_body, init)

        for g in range(NG):
            for i in range(K):
                outv[pl.ds(i * CW + g * L, L)] = state[g][i]
        pltpu.sync_copy(outv, out_hbm.at[wid])

    out = kmax(x)
    out = out.reshape(B, CB, K, CW).transpose(0, 1, 3, 2)
    return out.reshape(B, D * K)
